# baseline (speedup 1.0000x reference)
import jax
import jax.numpy as jnp
from jax import lax
from jax.experimental import pallas as pl
from jax.experimental.pallas import tpu as pltpu

_NUM_CODES = 8192
_DIM = 32
_TB = 256
_CH = 128
_LAG = 4


def _vq_body(x_ref, cb_ref, idx_smem, idx_ref, q_ref,
             b_ref, cbm2_ref, cbt_ref, m2_ref, a_ref):
    s = pl.program_id(0)
    nblocks = pl.num_programs(0) - _LAG - 1

    @pl.when(s == 0)
    def _():
        cb0 = cb_ref[...]
        b_ref[...] = jnp.sum(cb0 * cb0, axis=0, keepdims=True)
        cbm2_ref[...] = -2.0 * cb0
        cbt_ref[...] = cb0.T

    @pl.when(s < nblocks)
    def _():
        ph = (s % 2) * _TB
        x = x_ref[...]
        a_ref[pl.ds(ph, _TB), :] = jnp.sum(x * x, axis=1, keepdims=True)
        m2_ref[pl.ds(ph, _TB), :] = jnp.dot(
            x, cbm2_ref[...], preferred_element_type=jnp.float32)

    def _gather_tokens(base, count):
        for u in range(count):
            i = idx_smem[0, 0, base + u]
            q_ref[pl.ds(base + u, 1), :] = cbt_ref[pl.ds(i, 1), :]

    @pl.when((s >= 1) & (s <= nblocks))
    def _():
        ph = ((s - 1) % 2) * _TB
        a = a_ref[pl.ds(ph, _TB), :]
        b = b_ref[...]

        nchunks = _NUM_CODES // _CH
        per_chunk = _TB // nchunks
        run_min = jnp.full((_TB, _CH), jnp.inf, dtype=jnp.float32)
        run_chunk = jnp.zeros((_TB, _CH), dtype=jnp.int32)
        for j in range(nchunks):
            nrm = (a + b[:, j * _CH:(j + 1) * _CH]) \
                + m2_ref[pl.ds(ph, _TB), j * _CH:(j + 1) * _CH]
            lt = nrm < run_min
            run_min = jnp.where(lt, nrm, run_min)
            run_chunk = jnp.where(lt, j, run_chunk)
            _gather_tokens(j * per_chunk, per_chunk)

        gmin = jnp.min(run_min, axis=1, keepdims=True)
        lane = jax.lax.broadcasted_iota(jnp.int32, (_TB, _CH), 1)
        cand = run_chunk * _CH + lane
        idx = jnp.min(jnp.where(run_min == gmin, cand, _NUM_CODES), axis=1)
        idx_ref[...] = idx.reshape(1, 1, _TB)

    @pl.when(s > nblocks)
    def _():
        def outer(t8, carry):
            _gather_tokens(t8 * 8, 8)
            return carry
        lax.fori_loop(0, _TB // 8, outer, 0)


def kernel(inputs, codebook):
    original_shape = inputs.shape
    x = inputs.reshape(-1, _DIM)
    n = x.shape[0]
    nblocks = n // _TB
    nsteps = nblocks + _LAG + 1
    idx_seed = jnp.zeros((nblocks + 1, 1, _TB), jnp.int32)
    _, q = pl.pallas_call(
        _vq_body,
        grid=(nsteps,),
        in_specs=[
            pl.BlockSpec((_TB, _DIM), lambda s: (jnp.minimum(s, nblocks - 1), 0)),
            pl.BlockSpec((_DIM, _NUM_CODES), lambda s: (0, 0)),
            pl.BlockSpec((1, 1, _TB),
                         lambda s: (jnp.where(s >= _LAG + 1, s - _LAG - 1, nblocks),
                                    0, 0),
                         memory_space=pltpu.SMEM),
        ],
        out_specs=[
            pl.BlockSpec((1, 1, _TB),
                         lambda s: (jnp.where(s <= nblocks,
                                              jnp.maximum(s - 1, 0), nblocks),
                                    0, 0)),
            pl.BlockSpec((_TB, _DIM),
                         lambda s: (jnp.maximum(s - _LAG - 1, 0), 0)),
        ],
        out_shape=(
            jax.ShapeDtypeStruct((nblocks + 1, 1, _TB), jnp.int32),
            jax.ShapeDtypeStruct((n, _DIM), jnp.float32),
        ),
        scratch_shapes=[
            pltpu.VMEM((1, _NUM_CODES), jnp.float32),
            pltpu.VMEM((_DIM, _NUM_CODES), jnp.float32),
            pltpu.VMEM((_NUM_CODES, _DIM), jnp.float32),
            pltpu.VMEM((2 * _TB, _NUM_CODES), jnp.float32),
            pltpu.VMEM((2 * _TB, 1), jnp.float32),
        ],
        input_output_aliases={2: 0},
    )(x, codebook, idx_seed)
    return q.reshape(original_shape)


# speedup vs baseline: 1.3291x; 1.3291x over previous
import jax
import jax.numpy as jnp
from jax import lax
from jax.experimental import pallas as pl
from jax.experimental.pallas import tpu as pltpu

_NUM_CODES = 8192
_DIM = 32
_TB = 512
_CH = 128
_LAG = 4


def _vq_body(x_ref, cb_ref, idx_smem, idx_ref, q_ref, b_ref, cbm2_ref, cbt_ref):
    s = pl.program_id(0)
    nblocks = pl.num_programs(0) - _LAG

    @pl.when(s == 0)
    def _():
        cb0 = cb_ref[...]
        b_ref[...] = jnp.sum(cb0 * cb0, axis=0, keepdims=True)
        cbm2_ref[...] = -2.0 * cb0
        cbt_ref[...] = cb0.T

    def _gather_tokens(base, count):
        for u in range(count):
            i = idx_smem[0, 0, base + u]
            q_ref[pl.ds(base + u, 1), :] = cbt_ref[pl.ds(i, 1), :]

    @pl.when(s < nblocks)
    def _():
        x = x_ref[...]
        a = jnp.sum(x * x, axis=1, keepdims=True)
        m2 = jnp.dot(x, cbm2_ref[...], preferred_element_type=jnp.float32)
        b = b_ref[...]

        nchunks = _NUM_CODES // _CH
        per_chunk = _TB // nchunks
        run_min = jnp.full((_TB, _CH), jnp.inf, dtype=jnp.float32)
        run_chunk = jnp.zeros((_TB, _CH), dtype=jnp.int32)
        for j in range(nchunks):
            sl = slice(j * _CH, (j + 1) * _CH)
            nrm = (a + b[:, sl]) + m2[:, sl]
            lt = nrm < run_min
            run_min = jnp.where(lt, nrm, run_min)
            run_chunk = jnp.where(lt, j, run_chunk)
            _gather_tokens(j * per_chunk, per_chunk)

        gmin = jnp.min(run_min, axis=1, keepdims=True)
        lane = jax.lax.broadcasted_iota(jnp.int32, (_TB, _CH), 1)
        cand = run_chunk * _CH + lane
        idx = jnp.min(jnp.where(run_min == gmin, cand, _NUM_CODES), axis=1)
        idx_ref[...] = idx.reshape(1, 1, _TB)

    @pl.when(s >= nblocks)
    def _():
        def outer(t8, carry):
            _gather_tokens(t8 * 8, 8)
            return carry
        lax.fori_loop(0, _TB // 8, outer, 0)


def kernel(inputs, codebook):
    original_shape = inputs.shape
    x = inputs.reshape(-1, _DIM)
    n = x.shape[0]
    nblocks = n // _TB
    nsteps = nblocks + _LAG
    idx_seed = jnp.zeros((nblocks + 1, 1, _TB), jnp.int32)
    _, q = pl.pallas_call(
        _vq_body,
        grid=(nsteps,),
        in_specs=[
            pl.BlockSpec((_TB, _DIM), lambda s: (jnp.minimum(s, nblocks - 1), 0)),
            pl.BlockSpec((_DIM, _NUM_CODES), lambda s: (0, 0)),
            pl.BlockSpec((1, 1, _TB),
                         lambda s: (jnp.where(s >= _LAG, s - _LAG, nblocks), 0, 0),
                         memory_space=pltpu.SMEM),
        ],
        out_specs=[
            pl.BlockSpec((1, 1, _TB), lambda s: (jnp.minimum(s, nblocks), 0, 0)),
            pl.BlockSpec((_TB, _DIM), lambda s: (jnp.maximum(s - _LAG, 0), 0)),
        ],
        out_shape=(
            jax.ShapeDtypeStruct((nblocks + 1, 1, _TB), jnp.int32),
            jax.ShapeDtypeStruct((n, _DIM), jnp.float32),
        ),
        scratch_shapes=[
            pltpu.VMEM((1, _NUM_CODES), jnp.float32),
            pltpu.VMEM((_DIM, _NUM_CODES), jnp.float32),
            pltpu.VMEM((_NUM_CODES, _DIM), jnp.float32),
        ],
        input_output_aliases={2: 0},
    )(x, codebook, idx_seed)
    return q.reshape(original_shape)


# speedup vs baseline: 1.3595x; 1.0228x over previous
import jax
import jax.numpy as jnp
from jax import lax
from jax.experimental import pallas as pl
from jax.experimental.pallas import tpu as pltpu

_NUM_CODES = 8192
_DIM = 32
_TB = 256
_CH = 128
_LAG = 4


def _vq_body(x_ref, cb_ref, idx_smem, idx_ref, q_ref, b_ref, cbm2_ref, cbt_ref):
    s = pl.program_id(0)
    nblocks = pl.num_programs(0) - _LAG

    @pl.when(s == 0)
    def _():
        cb0 = cb_ref[...]
        b_ref[...] = jnp.sum(cb0 * cb0, axis=0, keepdims=True)
        cbm2_ref[...] = -2.0 * cb0
        cbt_ref[...] = cb0.T

    def _gather_tokens(base, count):
        for u in range(count):
            i = idx_smem[0, 0, base + u]
            q_ref[pl.ds(base + u, 1), :] = cbt_ref[pl.ds(i, 1), :]

    @pl.when(s < nblocks)
    def _():
        x = x_ref[...]
        a = jnp.sum(x * x, axis=1, keepdims=True)
        m2 = jnp.dot(x, cbm2_ref[...], preferred_element_type=jnp.float32)
        b = b_ref[...]

        nchunks = _NUM_CODES // _CH
        per_chunk = _TB // nchunks
        run_min = jnp.full((_TB, _CH), jnp.inf, dtype=jnp.float32)
        run_chunk = jnp.zeros((_TB, _CH), dtype=jnp.int32)
        for j in range(nchunks):
            sl = slice(j * _CH, (j + 1) * _CH)
            nrm = (a + b[:, sl]) + m2[:, sl]
            lt = nrm < run_min
            run_min = jnp.where(lt, nrm, run_min)
            run_chunk = jnp.where(lt, j, run_chunk)
            _gather_tokens(j * per_chunk, per_chunk)

        gmin = jnp.min(run_min, axis=1, keepdims=True)
        lane = jax.lax.broadcasted_iota(jnp.int32, (_TB, _CH), 1)
        cand = run_chunk * _CH + lane
        idx = jnp.min(jnp.where(run_min == gmin, cand, _NUM_CODES), axis=1)
        idx_ref[...] = idx.reshape(1, 1, _TB)

    @pl.when(s >= nblocks)
    def _():
        def outer(t8, carry):
            _gather_tokens(t8 * 8, 8)
            return carry
        lax.fori_loop(0, _TB // 8, outer, 0)


def kernel(inputs, codebook):
    original_shape = inputs.shape
    x = inputs.reshape(-1, _DIM)
    n = x.shape[0]
    nblocks = n // _TB
    nsteps = nblocks + _LAG
    idx_seed = jnp.zeros((nblocks + 1, 1, _TB), jnp.int32)
    _, q = pl.pallas_call(
        _vq_body,
        grid=(nsteps,),
        in_specs=[
            pl.BlockSpec((_TB, _DIM), lambda s: (jnp.minimum(s, nblocks - 1), 0)),
            pl.BlockSpec((_DIM, _NUM_CODES), lambda s: (0, 0)),
            pl.BlockSpec((1, 1, _TB),
                         lambda s: (jnp.where(s >= _LAG, s - _LAG, nblocks), 0, 0),
                         memory_space=pltpu.SMEM),
        ],
        out_specs=[
            pl.BlockSpec((1, 1, _TB), lambda s: (jnp.minimum(s, nblocks), 0, 0)),
            pl.BlockSpec((_TB, _DIM), lambda s: (jnp.maximum(s - _LAG, 0), 0)),
        ],
        out_shape=(
            jax.ShapeDtypeStruct((nblocks + 1, 1, _TB), jnp.int32),
            jax.ShapeDtypeStruct((n, _DIM), jnp.float32),
        ),
        scratch_shapes=[
            pltpu.VMEM((1, _NUM_CODES), jnp.float32),
            pltpu.VMEM((_DIM, _NUM_CODES), jnp.float32),
            pltpu.VMEM((_NUM_CODES, _DIM), jnp.float32),
        ],
        input_output_aliases={2: 0},
    )(x, codebook, idx_seed)
    return q.reshape(original_shape)
